# Initial kernel scaffold; baseline (speedup 1.0000x reference)
import functools

import jax
import jax.numpy as jnp
from jax import lax
from jax.experimental import pallas as pl
from jax.experimental.pallas import tpu as pltpu
from jax.experimental.pallas import tpu_sc as plsc

N = 10000
E = 320000
D = 128
NPAD = 10240

NC = 2
NS = 16
L = 16

EPC = E // (NC * NS)
WROW = 80
NWIN = EPC // WROW
ECH = E // WROW
RPS = NPAD // NS

_MESH = plsc.VectorSubcoreMesh(core_axis_name="c", subcore_axis_name="s")

_f32 = jnp.float32


@functools.partial(
    pl.kernel,
    mesh=_MESH,
    out_type=(
        jax.ShapeDtypeStruct((NC, NPAD), _f32),
        jax.ShapeDtypeStruct((NC, NPAD), _f32),
    ),
    scratch_types=[
        pltpu.VMEM_SHARED((NPAD,), _f32),
        pltpu.VMEM_SHARED((NPAD,), _f32),
        pltpu.VMEM((NWIN, WROW), jnp.int32),
        pltpu.VMEM((NWIN, WROW), _f32),
        pltpu.VMEM((WROW,), _f32),
    ],
)
def _sc_degrees(dst_hbm, ew_hbm, z1_hbm, dego_hbm, degw_hbm,
                dego_sm, degw_sm, idxb, ewb, onesb):
    c = lax.axis_index("c")
    s = lax.axis_index("s")
    wid = c * NS + s

    @pl.loop(0, WROW // L)
    def _(i):
        onesb[pl.ds(i * L, L)] = jnp.full((L,), 1.0, _f32)

    pltpu.sync_copy(z1_hbm.at[pl.ds(s * RPS, RPS)],
                    dego_sm.at[pl.ds(s * RPS, RPS)])
    pltpu.sync_copy(z1_hbm.at[pl.ds(s * RPS, RPS)],
                    degw_sm.at[pl.ds(s * RPS, RPS)])
    pltpu.sync_copy(dst_hbm.at[pl.ds(wid * NWIN, NWIN)], idxb)
    pltpu.sync_copy(ew_hbm.at[pl.ds(wid * NWIN, NWIN)], ewb)
    plsc.subcore_barrier()

    @pl.loop(0, NWIN)
    def _(w):
        pltpu.sync_copy(onesb, dego_sm.at[idxb.at[w]], add=True)
        pltpu.sync_copy(ewb.at[w], degw_sm.at[idxb.at[w]], add=True)

    plsc.subcore_barrier()
    pltpu.sync_copy(dego_sm.at[pl.ds(s * RPS, RPS)],
                    dego_hbm.at[c, pl.ds(s * RPS, RPS)])
    pltpu.sync_copy(degw_sm.at[pl.ds(s * RPS, RPS)],
                    degw_hbm.at[c, pl.ds(s * RPS, RPS)])


def _make_sc_agg(with_ew):
    scratch = [
        pltpu.VMEM_SHARED((NPAD, D), _f32),
        pltpu.VMEM((NWIN, WROW), jnp.int32),
        pltpu.VMEM((NWIN, WROW), jnp.int32),
        pltpu.VMEM((WROW, D), _f32),
    ]
    if with_ew:
        scratch.append(pltpu.VMEM((NWIN, WROW), _f32))

    def body(h_hbm, src_hbm, dst_hbm, *rest):
        if with_ew:
            ew_hbm, z2_hbm, out_hbm, agg_sm, isb, idb, rows, ewb = rest
        else:
            z2_hbm, out_hbm, agg_sm, isb, idb, rows = rest
        c = lax.axis_index("c")
        s = lax.axis_index("s")
        wid = c * NS + s

        pltpu.sync_copy(z2_hbm.at[pl.ds(s * RPS, RPS)],
                        agg_sm.at[pl.ds(s * RPS, RPS)])
        pltpu.sync_copy(src_hbm.at[pl.ds(wid * NWIN, NWIN)], isb)
        pltpu.sync_copy(dst_hbm.at[pl.ds(wid * NWIN, NWIN)], idb)
        if with_ew:
            pltpu.sync_copy(ew_hbm.at[pl.ds(wid * NWIN, NWIN)], ewb)
        plsc.subcore_barrier()

        @pl.loop(0, NWIN)
        def _(w):
            pltpu.sync_copy(h_hbm.at[isb.at[w]], rows)
            if with_ew:
                @pl.loop(0, WROW)
                def _(k):
                    scale = plsc.load_gather(
                        ewb, [jnp.full((L,), w, jnp.int32),
                              jnp.full((L,), k, jnp.int32)])
                    for cc in range(D // L):
                        sl = (k, pl.ds(cc * L, L))
                        rows[sl] = rows[sl] * scale
            pltpu.sync_copy(rows, agg_sm.at[idb.at[w]], add=True)

        plsc.subcore_barrier()
        pltpu.sync_copy(agg_sm.at[pl.ds(s * RPS, RPS)],
                        out_hbm.at[c, pl.ds(s * RPS, RPS)])

    return pl.kernel(
        body,
        mesh=_MESH,
        out_type=jax.ShapeDtypeStruct((NC, NPAD, D), _f32),
        scratch_types=scratch,
    )


_sc_agg_plain = _make_sc_agg(with_ew=False)
_sc_agg_weighted = _make_sc_agg(with_ew=True)


BLK = 1024
GRID = NPAD // BLK

_row = pl.BlockSpec((BLK, D), lambda i: (i, 0))
_col = pl.BlockSpec((BLK, 1), lambda i: (i, 0))
_wgt = pl.BlockSpec((D, D), lambda i: (0, 0))
_bias = pl.BlockSpec((1, D), lambda i: (0, 0))


def _dot(a, b):
    return jnp.dot(a, b, preferred_element_type=_f32,
                   precision=lax.Precision.HIGHEST)


def _tc_matmul_body(x_ref, w_ref, o_ref):
    o_ref[...] = _dot(x_ref[...], w_ref[...])


_tc_matmul = pl.pallas_call(
    _tc_matmul_body,
    grid=(GRID,),
    in_specs=[_row, _wgt],
    out_specs=_row,
    out_shape=jax.ShapeDtypeStruct((NPAD, D), _f32),
)


def _tc_scale_body(p_ref, do0_ref, do1_ref, dw0_ref, dw1_ref,
                   hs_ref, d1_ref, d2_ref):
    d1 = lax.rsqrt(do0_ref[...] + do1_ref[...] + 1.0)
    d2 = lax.rsqrt(dw0_ref[...] + dw1_ref[...] + 1.0)
    hs_ref[...] = p_ref[...] * d1
    d1_ref[...] = d1
    d2_ref[...] = d2


_tc_scale = pl.pallas_call(
    _tc_scale_body,
    grid=(GRID,),
    in_specs=[_row, _col, _col, _col, _col],
    out_specs=[_row, _col, _col],
    out_shape=[
        jax.ShapeDtypeStruct((NPAD, D), _f32),
        jax.ShapeDtypeStruct((NPAD, 1), _f32),
        jax.ShapeDtypeStruct((NPAD, 1), _f32),
    ],
)


def _tc_layer2_body(a0_ref, a1_ref, hs1_ref, d1_ref, b1_ref, w_ref, d2_ref,
                    h_ref, hs2_ref):
    h = d1_ref[...] * (a0_ref[...] + a1_ref[...] + hs1_ref[...]) + b1_ref[...]
    h = jnp.maximum(h, 0.0)
    h_ref[...] = h
    hs2_ref[...] = _dot(h, w_ref[...]) * d2_ref[...]


_tc_layer2 = pl.pallas_call(
    _tc_layer2_body,
    grid=(GRID,),
    in_specs=[_row, _row, _row, _col, _bias, _wgt, _col],
    out_specs=[_row, _row],
    out_shape=[
        jax.ShapeDtypeStruct((NPAD, D), _f32),
        jax.ShapeDtypeStruct((NPAD, D), _f32),
    ],
)


def _tc_layer3_body(a0_ref, a1_ref, hs2_ref, d2_ref, bc_ref, h_ref, w_ref,
                    d1_ref, hs3_ref):
    hcrf = d2_ref[...] * (a0_ref[...] + a1_ref[...] + hs2_ref[...]) + bc_ref[...]
    h2 = 0.1 * h_ref[...] + hcrf
    hs3_ref[...] = _dot(h2, w_ref[...]) * d1_ref[...]


_tc_layer3 = pl.pallas_call(
    _tc_layer3_body,
    grid=(GRID,),
    in_specs=[_row, _row, _row, _col, _bias, _row, _wgt, _col],
    out_specs=_row,
    out_shape=jax.ShapeDtypeStruct((NPAD, D), _f32),
)


def _tc_final_body(a0_ref, a1_ref, hs3_ref, d1_ref, b2_ref, o_ref):
    o = d1_ref[...] * (a0_ref[...] + a1_ref[...] + hs3_ref[...]) + b2_ref[...]
    m = jnp.max(o, axis=1, keepdims=True)
    lse = jnp.log(jnp.sum(jnp.exp(o - m), axis=1, keepdims=True)) + m
    o_ref[...] = o - lse


_tc_final = pl.pallas_call(
    _tc_final_body,
    grid=(GRID,),
    in_specs=[_row, _row, _row, _col, _bias],
    out_specs=_row,
    out_shape=jax.ShapeDtypeStruct((NPAD, D), _f32),
)


def kernel(x, edge_index, edge_weight, W1, b1, Wc, bc, W2, b2):
    src = edge_index[0].astype(jnp.int32).reshape(ECH, WROW)
    dst = edge_index[1].astype(jnp.int32).reshape(ECH, WROW)
    ew = edge_weight.astype(_f32).reshape(ECH, WROW)

    xp = jnp.pad(x.astype(_f32), ((0, NPAD - N), (0, 0)))
    z1 = jnp.zeros((NPAD,), _f32)
    z2 = jnp.zeros((NPAD, D), _f32)
    b1r = b1.reshape(1, D)
    bcr = bc.reshape(1, D)
    b2r = b2.reshape(1, D)

    dego, degw = _sc_degrees(dst.reshape(E), ew.reshape(E), z1)
    dego = dego.reshape(NC, NPAD, 1)
    degw = degw.reshape(NC, NPAD, 1)
    p1 = _tc_matmul(xp, W1)
    hs1, dis1, dis2 = _tc_scale(p1, dego[0], dego[1], degw[0], degw[1])

    agg1 = _sc_agg_plain(hs1, src, dst, z2)
    h, hs2 = _tc_layer2(agg1[0], agg1[1], hs1, dis1, b1r, Wc, dis2)

    agg2 = _sc_agg_weighted(hs2, src, dst, ew, z2)
    hs3 = _tc_layer3(agg2[0], agg2[1], hs2, dis2, bcr, h, W2, dis1)

    agg3 = _sc_agg_plain(hs3, src, dst, z2)
    out = _tc_final(agg3[0], agg3[1], hs3, dis1, b2r)
    return out[:N]


# speedup vs baseline: 19.0458x; 19.0458x over previous
import dataclasses
import functools

import jax
import jax.numpy as jnp
from jax import lax
from jax.experimental import pallas as pl
from jax.experimental.pallas import tpu as pltpu
from jax.experimental.pallas import tpu_sc as plsc

N = 10000
E = 320000
D = 128
NPAD = 10240

NC = 2
NS = 16
L = 16

WROW = 128
EP = 327680
EPC = EP // (NC * NS)
NWIN = EPC // WROW
ECH = EP // WROW
RPS = NPAD // NS

_MESH = plsc.VectorSubcoreMesh(core_axis_name="c", subcore_axis_name="s")

_f32 = jnp.float32


@functools.partial(
    pl.kernel,
    mesh=_MESH,
    out_type=(
        jax.ShapeDtypeStruct((NC, NPAD), _f32),
        jax.ShapeDtypeStruct((NC, NPAD), _f32),
    ),
    scratch_types=[
        pltpu.VMEM_SHARED((NPAD,), _f32),
        pltpu.VMEM_SHARED((NPAD,), _f32),
        pltpu.VMEM((NWIN, WROW), jnp.int32),
        pltpu.VMEM((NWIN, WROW), _f32),
        pltpu.VMEM((WROW,), _f32),
    ],
)
def _sc_degrees(dst_hbm, ew_hbm, z1_hbm, dego_hbm, degw_hbm,
                dego_sm, degw_sm, idxb, ewb, onesb):
    c = lax.axis_index("c")
    s = lax.axis_index("s")
    wid = c * NS + s

    @pl.loop(0, WROW // L)
    def _(i):
        onesb[pl.ds(i * L, L)] = jnp.full((L,), 1.0, _f32)

    pltpu.sync_copy(z1_hbm.at[pl.ds(s * RPS, RPS)],
                    dego_sm.at[pl.ds(s * RPS, RPS)])
    pltpu.sync_copy(z1_hbm.at[pl.ds(s * RPS, RPS)],
                    degw_sm.at[pl.ds(s * RPS, RPS)])
    pltpu.sync_copy(dst_hbm.at[pl.ds(wid * NWIN, NWIN)], idxb)
    pltpu.sync_copy(ew_hbm.at[pl.ds(wid * NWIN, NWIN)], ewb)
    plsc.subcore_barrier()

    @pl.loop(0, NWIN)
    def _(w):
        pltpu.sync_copy(onesb, dego_sm.at[idxb.at[w]], add=True)
        pltpu.sync_copy(ewb.at[w], degw_sm.at[idxb.at[w]], add=True)

    plsc.subcore_barrier()
    pltpu.sync_copy(dego_sm.at[pl.ds(s * RPS, RPS)],
                    dego_hbm.at[c, pl.ds(s * RPS, RPS)])
    pltpu.sync_copy(degw_sm.at[pl.ds(s * RPS, RPS)],
                    degw_hbm.at[c, pl.ds(s * RPS, RPS)])


def _make_sc_agg(with_ew):
    scratch = [
        pltpu.VMEM_SHARED((NPAD, D), _f32),
        pltpu.VMEM((NWIN, WROW), jnp.int32),
        pltpu.VMEM((NWIN, WROW), jnp.int32),
        pltpu.VMEM((WROW, D), _f32),
    ]
    if with_ew:
        scratch.append(pltpu.VMEM((NWIN, WROW), _f32))

    def body(h_hbm, src_hbm, dst_hbm, *rest):
        if with_ew:
            ew_hbm, z2_hbm, out_hbm, agg_sm, isb, idb, rows, ewb = rest
        else:
            z2_hbm, out_hbm, agg_sm, isb, idb, rows = rest
        c = lax.axis_index("c")
        s = lax.axis_index("s")
        wid = c * NS + s

        pltpu.sync_copy(z2_hbm.at[pl.ds(s * RPS, RPS)],
                        agg_sm.at[pl.ds(s * RPS, RPS)])
        pltpu.sync_copy(src_hbm.at[pl.ds(wid * NWIN, NWIN)], isb)
        pltpu.sync_copy(dst_hbm.at[pl.ds(wid * NWIN, NWIN)], idb)
        if with_ew:
            pltpu.sync_copy(ew_hbm.at[pl.ds(wid * NWIN, NWIN)], ewb)
        plsc.subcore_barrier()

        @pl.loop(0, NWIN)
        def _(w):
            pltpu.sync_copy(h_hbm.at[isb.at[w]], rows)
            if with_ew:
                @pl.loop(0, WROW)
                def _(k):
                    scale = plsc.load_gather(
                        ewb, [jnp.full((L,), w, jnp.int32),
                              jnp.full((L,), k, jnp.int32)])
                    for cc in range(D // L):
                        sl = (k, pl.ds(cc * L, L))
                        rows[sl] = rows[sl] * scale
            pltpu.sync_copy(rows, agg_sm.at[idb.at[w]], add=True)

        plsc.subcore_barrier()
        pltpu.sync_copy(agg_sm.at[pl.ds(s * RPS, RPS)],
                        out_hbm.at[c, pl.ds(s * RPS, RPS)])

    cp = pltpu.CompilerParams()
    if with_ew and "needs_layout_passes" in pltpu.CompilerParams.__dataclass_fields__:
        cp = dataclasses.replace(cp, needs_layout_passes=False)
    return pl.kernel(
        body,
        mesh=_MESH,
        out_type=jax.ShapeDtypeStruct((NC, NPAD, D), _f32),
        scratch_types=scratch,
        compiler_params=cp,
    )


_sc_agg_plain = _make_sc_agg(with_ew=False)
_sc_agg_weighted = _make_sc_agg(with_ew=True)


BLK = 1024
GRID = NPAD // BLK

_row = pl.BlockSpec((BLK, D), lambda i: (i, 0))
_col = pl.BlockSpec((BLK, 1), lambda i: (i, 0))
_wgt = pl.BlockSpec((D, D), lambda i: (0, 0))
_bias = pl.BlockSpec((1, D), lambda i: (0, 0))


def _dot(a, b):
    return jnp.dot(a, b, preferred_element_type=_f32,
                   precision=lax.Precision.HIGHEST)


def _tc_matmul_body(x_ref, w_ref, o_ref):
    o_ref[...] = _dot(x_ref[...], w_ref[...])


_tc_matmul = pl.pallas_call(
    _tc_matmul_body,
    grid=(GRID,),
    in_specs=[_row, _wgt],
    out_specs=_row,
    out_shape=jax.ShapeDtypeStruct((NPAD, D), _f32),
)


def _tc_scale_body(p_ref, do0_ref, do1_ref, dw0_ref, dw1_ref,
                   hs_ref, d1_ref, d2_ref):
    d1 = lax.rsqrt(do0_ref[...] + do1_ref[...] + 1.0)
    d2 = lax.rsqrt(dw0_ref[...] + dw1_ref[...] + 1.0)
    hs_ref[...] = p_ref[...] * d1
    d1_ref[...] = d1
    d2_ref[...] = d2


_tc_scale = pl.pallas_call(
    _tc_scale_body,
    grid=(GRID,),
    in_specs=[_row, _col, _col, _col, _col],
    out_specs=[_row, _col, _col],
    out_shape=[
        jax.ShapeDtypeStruct((NPAD, D), _f32),
        jax.ShapeDtypeStruct((NPAD, 1), _f32),
        jax.ShapeDtypeStruct((NPAD, 1), _f32),
    ],
)


def _tc_layer2_body(a0_ref, a1_ref, hs1_ref, d1_ref, b1_ref, w_ref, d2_ref,
                    h_ref, hs2_ref):
    h = d1_ref[...] * (a0_ref[...] + a1_ref[...] + hs1_ref[...]) + b1_ref[...]
    h = jnp.maximum(h, 0.0)
    h_ref[...] = h
    hs2_ref[...] = _dot(h, w_ref[...]) * d2_ref[...]


_tc_layer2 = pl.pallas_call(
    _tc_layer2_body,
    grid=(GRID,),
    in_specs=[_row, _row, _row, _col, _bias, _wgt, _col],
    out_specs=[_row, _row],
    out_shape=[
        jax.ShapeDtypeStruct((NPAD, D), _f32),
        jax.ShapeDtypeStruct((NPAD, D), _f32),
    ],
)


def _tc_layer3_body(a0_ref, a1_ref, hs2_ref, d2_ref, bc_ref, h_ref, w_ref,
                    d1_ref, hs3_ref):
    hcrf = d2_ref[...] * (a0_ref[...] + a1_ref[...] + hs2_ref[...]) + bc_ref[...]
    h2 = 0.1 * h_ref[...] + hcrf
    hs3_ref[...] = _dot(h2, w_ref[...]) * d1_ref[...]


_tc_layer3 = pl.pallas_call(
    _tc_layer3_body,
    grid=(GRID,),
    in_specs=[_row, _row, _row, _col, _bias, _row, _wgt, _col],
    out_specs=_row,
    out_shape=jax.ShapeDtypeStruct((NPAD, D), _f32),
)


def _tc_final_body(a0_ref, a1_ref, hs3_ref, d1_ref, b2_ref, o_ref):
    o = d1_ref[...] * (a0_ref[...] + a1_ref[...] + hs3_ref[...]) + b2_ref[...]
    m = jnp.max(o, axis=1, keepdims=True)
    lse = jnp.log(jnp.sum(jnp.exp(o - m), axis=1, keepdims=True)) + m
    o_ref[...] = o - lse


_tc_final = pl.pallas_call(
    _tc_final_body,
    grid=(GRID,),
    in_specs=[_row, _row, _row, _col, _bias],
    out_specs=_row,
    out_shape=jax.ShapeDtypeStruct((NPAD, D), _f32),
)


def kernel(x, edge_index, edge_weight, W1, b1, Wc, bc, W2, b2):
    pad_idx = (jnp.arange(EP - E, dtype=jnp.int32) % (NPAD - N)) + N
    pad_w = jnp.zeros((EP - E,), _f32)
    src = jnp.concatenate(
        [edge_index[0].astype(jnp.int32), pad_idx]).reshape(ECH, WROW)
    dst = jnp.concatenate(
        [edge_index[1].astype(jnp.int32), pad_idx]).reshape(ECH, WROW)
    ew = jnp.concatenate(
        [edge_weight.astype(_f32), pad_w]).reshape(ECH, WROW)

    xp = jnp.pad(x.astype(_f32), ((0, NPAD - N), (0, 0)))
    z1 = jnp.zeros((NPAD,), _f32)
    z2 = jnp.zeros((NPAD, D), _f32)
    b1r = b1.reshape(1, D)
    bcr = bc.reshape(1, D)
    b2r = b2.reshape(1, D)

    dego, degw = _sc_degrees(dst, ew, z1)
    dego = dego.reshape(NC, NPAD, 1)
    degw = degw.reshape(NC, NPAD, 1)
    p1 = _tc_matmul(xp, W1)
    hs1, dis1, dis2 = _tc_scale(p1, dego[0], dego[1], degw[0], degw[1])

    agg1 = _sc_agg_plain(hs1, src, dst, z2)
    h, hs2 = _tc_layer2(agg1[0], agg1[1], hs1, dis1, b1r, Wc, dis2)

    agg2 = _sc_agg_weighted(hs2, src, dst, ew, z2)
    hs3 = _tc_layer3(agg2[0], agg2[1], hs2, dis2, bcr, h, W2, dis1)

    agg3 = _sc_agg_plain(hs3, src, dst, z2)
    out = _tc_final(agg3[0], agg3[1], hs3, dis1, b2r)
    return out[:N]


# speedup vs baseline: 27.7835x; 1.4588x over previous
import dataclasses
import functools

import jax
import jax.numpy as jnp
from jax import lax
from jax.experimental import pallas as pl
from jax.experimental.pallas import tpu as pltpu
from jax.experimental.pallas import tpu_sc as plsc

N = 10000
E = 320000
D = 128
NPAD = 10240

NC = 2
NS = 16
L = 16

WROW = 128
EP = 327680
EPC = EP // (NC * NS)
NWIN = EPC // WROW
ECH = EP // WROW
RPS = NPAD // NS

_MESH = plsc.VectorSubcoreMesh(core_axis_name="c", subcore_axis_name="s")

_f32 = jnp.float32


@functools.partial(
    pl.kernel,
    mesh=_MESH,
    out_type=(
        jax.ShapeDtypeStruct((NC, NPAD), _f32),
        jax.ShapeDtypeStruct((NC, NPAD), _f32),
    ),
    scratch_types=[
        pltpu.VMEM_SHARED((NPAD,), _f32),
        pltpu.VMEM_SHARED((NPAD,), _f32),
        pltpu.VMEM((NWIN, WROW), jnp.int32),
        pltpu.VMEM((NWIN, WROW), _f32),
        pltpu.VMEM((WROW,), _f32),
    ],
)
def _sc_degrees(dst_hbm, ew_hbm, z1_hbm, dego_hbm, degw_hbm,
                dego_sm, degw_sm, idxb, ewb, onesb):
    c = lax.axis_index("c")
    s = lax.axis_index("s")
    wid = c * NS + s

    @pl.loop(0, WROW // L)
    def _(i):
        onesb[pl.ds(i * L, L)] = jnp.full((L,), 1.0, _f32)

    pltpu.sync_copy(z1_hbm.at[pl.ds(s * RPS, RPS)],
                    dego_sm.at[pl.ds(s * RPS, RPS)])
    pltpu.sync_copy(z1_hbm.at[pl.ds(s * RPS, RPS)],
                    degw_sm.at[pl.ds(s * RPS, RPS)])
    pltpu.sync_copy(dst_hbm.at[pl.ds(wid * NWIN, NWIN)], idxb)
    pltpu.sync_copy(ew_hbm.at[pl.ds(wid * NWIN, NWIN)], ewb)
    plsc.subcore_barrier()

    @pl.loop(0, NWIN)
    def _(w):
        pltpu.sync_copy(onesb, dego_sm.at[idxb.at[w]], add=True)
        pltpu.sync_copy(ewb.at[w], degw_sm.at[idxb.at[w]], add=True)

    plsc.subcore_barrier()
    pltpu.sync_copy(dego_sm.at[pl.ds(s * RPS, RPS)],
                    dego_hbm.at[c, pl.ds(s * RPS, RPS)])
    pltpu.sync_copy(degw_sm.at[pl.ds(s * RPS, RPS)],
                    degw_hbm.at[c, pl.ds(s * RPS, RPS)])


CH = 8
NCH = NWIN // CH


def _make_sc_agg(with_ew):
    nset = 3 if with_ew else 2
    scratch = (
        [pltpu.VMEM_SHARED((NPAD, D), _f32)]
        + [pltpu.VMEM((CH, WROW), jnp.int32)] * 2
        + [pltpu.VMEM((CH, WROW), jnp.int32)] * 2
        + ([pltpu.VMEM((CH, WROW), _f32)] * 2 if with_ew else [])
        + [pltpu.VMEM((WROW, D), _f32)] * 2
        + [pltpu.SemaphoreType.DMA] * 2
    )

    def body(h_hbm, src_hbm, dst_hbm, *rest):
        if with_ew:
            (ew_hbm, z2_hbm, out_hbm, agg_sm, is_a, is_b, id_a, id_b,
             ew_a, ew_b, rows_a, rows_b, sem_a, sem_b) = rest
            idxsets = ((is_a, id_a, ew_a), (is_b, id_b, ew_b))
        else:
            (z2_hbm, out_hbm, agg_sm, is_a, is_b, id_a, id_b,
             rows_a, rows_b, sem_a, sem_b) = rest
            idxsets = ((is_a, id_a, None), (is_b, id_b, None))
        rowbuf = ((rows_a, sem_a), (rows_b, sem_b))
        c = lax.axis_index("c")
        s = lax.axis_index("s")
        wid = c * NS + s

        def load_idx(ch, iset):
            base = wid * NWIN + ch * CH
            pltpu.sync_copy(src_hbm.at[pl.ds(base, CH)], iset[0])
            pltpu.sync_copy(dst_hbm.at[pl.ds(base, CH)], iset[1])
            if with_ew:
                pltpu.sync_copy(ew_hbm.at[pl.ds(base, CH)], iset[2])

        def start_gather(iset, w, b):
            pltpu.async_copy(h_hbm.at[iset[0].at[w]], rowbuf[b][0],
                             rowbuf[b][1])

        def wait_gather(iset, w, b):
            pltpu.make_async_copy(h_hbm.at[iset[0].at[w]], rowbuf[b][0],
                                  rowbuf[b][1]).wait()

        def do_scatter(iset, w, b):
            buf = rowbuf[b][0]
            if with_ew:
                ewc = iset[2]

                @plsc.parallel_loop(0, WROW, unroll=4)
                def _(k):
                    scale = plsc.load_gather(
                        ewc, [jnp.full((L,), w, jnp.int32),
                              jnp.full((L,), k, jnp.int32)])
                    for cc in range(D // L):
                        sl = (k, pl.ds(cc * L, L))
                        buf[sl] = buf[sl] * scale
            pltpu.sync_copy(buf, agg_sm.at[iset[1].at[w]], add=True)

        def do_chunk(ch, cur, nxt):
            @pl.when(ch + 1 < NCH)
            def _():
                load_idx(ch + 1, idxsets[nxt])
            for w in range(CH):
                b = w % 2
                if w + 1 < CH:
                    start_gather(idxsets[cur], w + 1, 1 - b)
                else:
                    @pl.when(ch + 1 < NCH)
                    def _():
                        start_gather(idxsets[nxt], 0, 1 - b)
                wait_gather(idxsets[cur], w, b)
                do_scatter(idxsets[cur], w, b)

        pltpu.sync_copy(z2_hbm.at[pl.ds(s * RPS, RPS)],
                        agg_sm.at[pl.ds(s * RPS, RPS)])
        plsc.subcore_barrier()

        load_idx(0, idxsets[0])
        start_gather(idxsets[0], 0, 0)

        @pl.loop(0, NCH // 2)
        def _(q):
            do_chunk(2 * q, 0, 1)
            do_chunk(2 * q + 1, 1, 0)

        plsc.subcore_barrier()
        pltpu.sync_copy(agg_sm.at[pl.ds(s * RPS, RPS)],
                        out_hbm.at[c, pl.ds(s * RPS, RPS)])

    cp = pltpu.CompilerParams()
    if with_ew and "needs_layout_passes" in pltpu.CompilerParams.__dataclass_fields__:
        cp = dataclasses.replace(cp, needs_layout_passes=False)
    return pl.kernel(
        body,
        mesh=_MESH,
        out_type=jax.ShapeDtypeStruct((NC, NPAD, D), _f32),
        scratch_types=scratch,
        compiler_params=cp,
    )


_sc_agg_plain = _make_sc_agg(with_ew=False)
_sc_agg_weighted = _make_sc_agg(with_ew=True)


BLK = 1024
GRID = NPAD // BLK

_row = pl.BlockSpec((BLK, D), lambda i: (i, 0))
_col = pl.BlockSpec((BLK, 1), lambda i: (i, 0))
_wgt = pl.BlockSpec((D, D), lambda i: (0, 0))
_bias = pl.BlockSpec((1, D), lambda i: (0, 0))


def _dot(a, b):
    return jnp.dot(a, b, preferred_element_type=_f32,
                   precision=lax.Precision.HIGHEST)


def _tc_matmul_body(x_ref, w_ref, o_ref):
    o_ref[...] = _dot(x_ref[...], w_ref[...])


_tc_matmul = pl.pallas_call(
    _tc_matmul_body,
    grid=(GRID,),
    in_specs=[_row, _wgt],
    out_specs=_row,
    out_shape=jax.ShapeDtypeStruct((NPAD, D), _f32),
)


def _tc_scale_body(p_ref, do0_ref, do1_ref, dw0_ref, dw1_ref,
                   hs_ref, d1_ref, d2_ref):
    d1 = lax.rsqrt(do0_ref[...] + do1_ref[...] + 1.0)
    d2 = lax.rsqrt(dw0_ref[...] + dw1_ref[...] + 1.0)
    hs_ref[...] = p_ref[...] * d1
    d1_ref[...] = d1
    d2_ref[...] = d2


_tc_scale = pl.pallas_call(
    _tc_scale_body,
    grid=(GRID,),
    in_specs=[_row, _col, _col, _col, _col],
    out_specs=[_row, _col, _col],
    out_shape=[
        jax.ShapeDtypeStruct((NPAD, D), _f32),
        jax.ShapeDtypeStruct((NPAD, 1), _f32),
        jax.ShapeDtypeStruct((NPAD, 1), _f32),
    ],
)


def _tc_layer2_body(a0_ref, a1_ref, hs1_ref, d1_ref, b1_ref, w_ref, d2_ref,
                    h_ref, hs2_ref):
    h = d1_ref[...] * (a0_ref[...] + a1_ref[...] + hs1_ref[...]) + b1_ref[...]
    h = jnp.maximum(h, 0.0)
    h_ref[...] = h
    hs2_ref[...] = _dot(h, w_ref[...]) * d2_ref[...]


_tc_layer2 = pl.pallas_call(
    _tc_layer2_body,
    grid=(GRID,),
    in_specs=[_row, _row, _row, _col, _bias, _wgt, _col],
    out_specs=[_row, _row],
    out_shape=[
        jax.ShapeDtypeStruct((NPAD, D), _f32),
        jax.ShapeDtypeStruct((NPAD, D), _f32),
    ],
)


def _tc_layer3_body(a0_ref, a1_ref, hs2_ref, d2_ref, bc_ref, h_ref, w_ref,
                    d1_ref, hs3_ref):
    hcrf = d2_ref[...] * (a0_ref[...] + a1_ref[...] + hs2_ref[...]) + bc_ref[...]
    h2 = 0.1 * h_ref[...] + hcrf
    hs3_ref[...] = _dot(h2, w_ref[...]) * d1_ref[...]


_tc_layer3 = pl.pallas_call(
    _tc_layer3_body,
    grid=(GRID,),
    in_specs=[_row, _row, _row, _col, _bias, _row, _wgt, _col],
    out_specs=_row,
    out_shape=jax.ShapeDtypeStruct((NPAD, D), _f32),
)


def _tc_final_body(a0_ref, a1_ref, hs3_ref, d1_ref, b2_ref, o_ref):
    o = d1_ref[...] * (a0_ref[...] + a1_ref[...] + hs3_ref[...]) + b2_ref[...]
    m = jnp.max(o, axis=1, keepdims=True)
    lse = jnp.log(jnp.sum(jnp.exp(o - m), axis=1, keepdims=True)) + m
    o_ref[...] = o - lse


_tc_final = pl.pallas_call(
    _tc_final_body,
    grid=(GRID,),
    in_specs=[_row, _row, _row, _col, _bias],
    out_specs=_row,
    out_shape=jax.ShapeDtypeStruct((NPAD, D), _f32),
)


def kernel(x, edge_index, edge_weight, W1, b1, Wc, bc, W2, b2):
    pad_idx = (jnp.arange(EP - E, dtype=jnp.int32) % (NPAD - N)) + N
    pad_w = jnp.zeros((EP - E,), _f32)
    src = jnp.concatenate(
        [edge_index[0].astype(jnp.int32), pad_idx]).reshape(ECH, WROW)
    dst = jnp.concatenate(
        [edge_index[1].astype(jnp.int32), pad_idx]).reshape(ECH, WROW)
    ew = jnp.concatenate(
        [edge_weight.astype(_f32), pad_w]).reshape(ECH, WROW)

    xp = jnp.pad(x.astype(_f32), ((0, NPAD - N), (0, 0)))
    z1 = jnp.zeros((NPAD,), _f32)
    z2 = jnp.zeros((NPAD, D), _f32)
    b1r = b1.reshape(1, D)
    bcr = bc.reshape(1, D)
    b2r = b2.reshape(1, D)

    dego, degw = _sc_degrees(dst, ew, z1)
    dego = dego.reshape(NC, NPAD, 1)
    degw = degw.reshape(NC, NPAD, 1)
    p1 = _tc_matmul(xp, W1)
    hs1, dis1, dis2 = _tc_scale(p1, dego[0], dego[1], degw[0], degw[1])

    agg1 = _sc_agg_plain(hs1, src, dst, z2)
    h, hs2 = _tc_layer2(agg1[0], agg1[1], hs1, dis1, b1r, Wc, dis2)

    agg2 = _sc_agg_weighted(hs2, src, dst, ew, z2)
    hs3 = _tc_layer3(agg2[0], agg2[1], hs2, dis2, bcr, h, W2, dis1)

    agg3 = _sc_agg_plain(hs3, src, dst, z2)
    out = _tc_final(agg3[0], agg3[1], hs3, dis1, b2r)
    return out[:N]
